# baseline (speedup 1.0000x reference)
import jax
import jax.numpy as jnp
from jax.experimental import pallas as pl
from jax.experimental.pallas import tpu as pltpu

_EPS = 1e-05
_BN = 4096
_BC = 256


def _bn_kernel(x_ref, m_ref, g_ref, b_ref, mu_ref, var_ref, o_ref):
    inv = jax.lax.rsqrt(var_ref[...] + _EPS)
    scale = g_ref[...] * inv
    bias = b_ref[...] - mu_ref[...] * scale
    x = x_ref[...]
    m = m_ref[...]
    normed = x * scale + bias
    o_ref[...] = x + m * (normed - x)


def kernel(x_flat_nc, mask_flat, gamma, beta, moving_mean, moving_var):
    n, c = x_flat_nc.shape
    m2d = mask_flat.astype(jnp.float32)[:, None]
    grid = (n // _BN, c // _BC)
    return pl.pallas_call(
        _bn_kernel,
        grid=grid,
        in_specs=[
            pl.BlockSpec((_BN, _BC), lambda i, j: (i, j)),
            pl.BlockSpec((_BN, 1), lambda i, j: (i, 0)),
            pl.BlockSpec((1, _BC), lambda i, j: (0, j)),
            pl.BlockSpec((1, _BC), lambda i, j: (0, j)),
            pl.BlockSpec((1, _BC), lambda i, j: (0, j)),
            pl.BlockSpec((1, _BC), lambda i, j: (0, j)),
        ],
        out_specs=pl.BlockSpec((_BN, _BC), lambda i, j: (i, j)),
        out_shape=jax.ShapeDtypeStruct((n, c), x_flat_nc.dtype),
    )(x_flat_nc, m2d, gamma[None, :], beta[None, :],
      moving_mean[None, :], moving_var[None, :])
